# baseline (speedup 1.0000x reference)
import functools

import jax
import jax.numpy as jnp
from jax import lax
from jax.experimental import pallas as pl
from jax.experimental.pallas import tpu as pltpu
from jax.experimental.pallas import tpu_sc as plsc

BATCH = 16384
DIM = 32
PACK = 128 // DIM
NW = 32
B_PER_W = BATCH // NW
N_CHUNK = 4
CHUNK = B_PER_W // N_CHUNK
BLOCKS_PER_CHUNK = CHUNK // 16


def _body(hi_u_hbm, hi_v_hbm, lo_u_hbm, lo_v_hbm, u_hbm, v_hbm, out_hbm,
          idx_u, idx_v, lo_u, lo_v, bu0, bu1, bv0, bv1, out_v,
          su0, su1, sv0, sv1):
  wid = lax.axis_index("s") * 2 + lax.axis_index("c")
  base = wid * B_PER_W

  pltpu.sync_copy(hi_u_hbm.at[pl.ds(wid * N_CHUNK, N_CHUNK)], idx_u)
  pltpu.sync_copy(hi_v_hbm.at[pl.ds(wid * N_CHUNK, N_CHUNK)], idx_v)
  pltpu.sync_copy(lo_u_hbm.at[pl.ds(base, B_PER_W)], lo_u)
  pltpu.sync_copy(lo_v_hbm.at[pl.ds(base, B_PER_W)], lo_v)

  bufs_u = (bu0, bu1)
  bufs_v = (bv0, bv1)
  sems_u = (su0, su1)
  sems_v = (sv0, sv1)
  iota = lax.iota(jnp.int32, 16)

  def start(j):
    b = j % 2
    du = pltpu.async_copy(u_hbm.at[idx_u.at[j]], bufs_u[b], sems_u[b])
    dv = pltpu.async_copy(v_hbm.at[idx_v.at[j]], bufs_v[b], sems_v[b])
    return du, dv

  descs = {0: start(0)}
  for j in range(N_CHUNK):
    if j + 1 < N_CHUNK:
      descs[j + 1] = start(j + 1)
    du, dv = descs.pop(j)
    du.wait()
    dv.wait()
    b = j % 2
    bu, bv = bufs_u[b], bufs_v[b]

    def block(k, _):
      rows16 = k * 16 + iota
      glob = j * CHUNK + rows16
      off_u = plsc.load_gather(lo_u, [glob])
      off_v = plsc.load_gather(lo_v, [glob])
      accs = [jnp.zeros((16,), jnp.float32) for _ in range(4)]
      for d in range(DIM):
        ug = plsc.load_gather(bu, [rows16, off_u + d])
        vg = plsc.load_gather(bv, [rows16, off_v + d])
        accs[d % 4] = accs[d % 4] + ug * vg
      acc = (accs[0] + accs[1]) + (accs[2] + accs[3])
      plsc.store_scatter(out_v, [glob], acc)
      return ()

    lax.fori_loop(0, BLOCKS_PER_CHUNK, block, (), unroll=False)

  pltpu.sync_copy(out_v, out_hbm.at[pl.ds(base, B_PER_W)])


@functools.partial(
    pl.kernel,
    out_type=jax.ShapeDtypeStruct((BATCH,), jnp.float32),
    mesh=plsc.VectorSubcoreMesh(core_axis_name="c", subcore_axis_name="s"),
    compiler_params=pltpu.CompilerParams(
        needs_layout_passes=False, use_tc_tiling_on_sc=False),
    scratch_types=[
        pltpu.VMEM((N_CHUNK, CHUNK), jnp.int32),
        pltpu.VMEM((N_CHUNK, CHUNK), jnp.int32),
        pltpu.VMEM((B_PER_W,), jnp.int32),
        pltpu.VMEM((B_PER_W,), jnp.int32),
        pltpu.VMEM((CHUNK, 128), jnp.float32),
        pltpu.VMEM((CHUNK, 128), jnp.float32),
        pltpu.VMEM((CHUNK, 128), jnp.float32),
        pltpu.VMEM((CHUNK, 128), jnp.float32),
        pltpu.VMEM((B_PER_W,), jnp.float32),
        pltpu.SemaphoreType.DMA,
        pltpu.SemaphoreType.DMA,
        pltpu.SemaphoreType.DMA,
        pltpu.SemaphoreType.DMA,
    ],
)
def _mf_sc(*refs):
  _body(*refs)


def kernel(x, U, V):
  xu = x[:, 0]
  xv = x[:, 1]
  hi_u = (xu >> 2).reshape(BATCH // CHUNK, CHUNK)
  hi_v = (xv >> 2).reshape(BATCH // CHUNK, CHUNK)
  lo_u = (xu & 3) << 5
  lo_v = (xv & 3) << 5
  U2 = U.reshape(U.shape[0] // PACK, 128)
  V2 = V.reshape(V.shape[0] // PACK, 128)
  return _mf_sc(hi_u, hi_v, lo_u, lo_v, U2, V2)


# speedup vs baseline: 1.0020x; 1.0020x over previous
import functools

import jax
import jax.numpy as jnp
from jax import lax
from jax.experimental import pallas as pl
from jax.experimental.pallas import tpu as pltpu
from jax.experimental.pallas import tpu_sc as plsc

BATCH = 16384
DIM = 32
PACK = 128 // DIM
NW = 32
B_PER_W = BATCH // NW
N_CHUNK = 4
CHUNK = B_PER_W // N_CHUNK
BLOCKS_PER_CHUNK = CHUNK // 16


def _body(hi_u_hbm, hi_v_hbm, lo_u_hbm, lo_v_hbm, u_hbm, v_hbm, out_hbm,
          idx_u, idx_v, lo_u, lo_v, bu0, bu1, bv0, bv1, out_v,
          su0, su1, sv0, sv1):
  wid = lax.axis_index("s") * 2 + lax.axis_index("c")
  base = wid * B_PER_W

  pltpu.sync_copy(hi_u_hbm.at[pl.ds(wid * N_CHUNK, N_CHUNK)], idx_u)
  pltpu.sync_copy(hi_v_hbm.at[pl.ds(wid * N_CHUNK, N_CHUNK)], idx_v)
  pltpu.sync_copy(lo_u_hbm.at[pl.ds(base, B_PER_W)], lo_u)
  pltpu.sync_copy(lo_v_hbm.at[pl.ds(base, B_PER_W)], lo_v)

  bufs_u = (bu0, bu1)
  bufs_v = (bv0, bv1)
  sems_u = (su0, su1)
  sems_v = (sv0, sv1)
  iota = lax.iota(jnp.int32, 16)

  def start(j):
    b = j % 2
    du = pltpu.async_copy(u_hbm.at[idx_u.at[j]], bufs_u[b], sems_u[b])
    dv = pltpu.async_copy(v_hbm.at[idx_v.at[j]], bufs_v[b], sems_v[b])
    return du, dv

  descs = {0: start(0)}
  for j in range(N_CHUNK):
    if j + 1 < N_CHUNK:
      descs[j + 1] = start(j + 1)
    du, dv = descs.pop(j)
    du.wait()
    dv.wait()
    b = j % 2
    bu, bv = bufs_u[b], bufs_v[b]

    def block(k, _):
      rows16 = k * 16 + iota
      glob = j * CHUNK + rows16
      off_u = plsc.load_gather(lo_u, [glob])
      off_v = plsc.load_gather(lo_v, [glob])
      accs = [jnp.zeros((16,), jnp.float32) for _ in range(4)]
      for d in range(DIM):
        ug = plsc.load_gather(bu, [rows16, off_u + d])
        vg = plsc.load_gather(bv, [rows16, off_v + d])
        accs[d % 4] = accs[d % 4] + ug * vg
      acc = (accs[0] + accs[1]) + (accs[2] + accs[3])
      plsc.store_scatter(out_v, [glob], acc)
      return ()

    lax.fori_loop(0, BLOCKS_PER_CHUNK, block, (), unroll=False)

  pltpu.sync_copy(out_v, out_hbm.at[pl.ds(base, B_PER_W)])


@functools.partial(
    pl.kernel,
    out_type=jax.ShapeDtypeStruct((BATCH,), jnp.float32),
    mesh=plsc.VectorSubcoreMesh(core_axis_name="c", subcore_axis_name="s"),
    compiler_params=pltpu.CompilerParams(
        needs_layout_passes=False, use_tc_tiling_on_sc=True),
    scratch_types=[
        pltpu.VMEM((N_CHUNK, CHUNK), jnp.int32),
        pltpu.VMEM((N_CHUNK, CHUNK), jnp.int32),
        pltpu.VMEM((B_PER_W,), jnp.int32),
        pltpu.VMEM((B_PER_W,), jnp.int32),
        pltpu.VMEM((CHUNK, 128), jnp.float32),
        pltpu.VMEM((CHUNK, 128), jnp.float32),
        pltpu.VMEM((CHUNK, 128), jnp.float32),
        pltpu.VMEM((CHUNK, 128), jnp.float32),
        pltpu.VMEM((B_PER_W,), jnp.float32),
        pltpu.SemaphoreType.DMA,
        pltpu.SemaphoreType.DMA,
        pltpu.SemaphoreType.DMA,
        pltpu.SemaphoreType.DMA,
    ],
)
def _mf_sc(*refs):
  _body(*refs)


def kernel(x, U, V):
  xu = x[:, 0]
  xv = x[:, 1]
  hi_u = (xu >> 2).reshape(BATCH // CHUNK, CHUNK)
  hi_v = (xv >> 2).reshape(BATCH // CHUNK, CHUNK)
  lo_u = (xu & 3) << 5
  lo_v = (xv & 3) << 5
  U2 = U.reshape(U.shape[0] // PACK, 128)
  V2 = V.reshape(V.shape[0] // PACK, 128)
  return _mf_sc(hi_u, hi_v, lo_u, lo_v, U2, V2)
